# Initial kernel scaffold; baseline (speedup 1.0000x reference)
import jax
import jax.numpy as jnp
from jax.experimental import pallas as pl


def kernel(x, t, graph_sizes, W_node, b_node, W1, b1, W2, b2):
    raise NotImplementedError("write your pallas kernel here")


# speedup vs baseline: 2.1455x; 2.1455x over previous
import functools
import math

import numpy as np
import jax
import jax.numpy as jnp
from jax import lax
from jax.experimental import pallas as pl
from jax.experimental.pallas import tpu as pltpu
from jax.experimental.pallas import tpu_sc as plsc

B = 8
PER = 256
N = B * PER
FEAT = 512
TDIM = 256
TRI = PER * (PER - 1) // 2
WORKERS_PER_GRAPH = 4
NUM_WORKERS = 32
OUT_PER_W = TRI // WORKERS_PER_GRAPH
CHUNKS = OUT_PER_W // 16
WIN_ROWS = 128


def _build_triu_tables():
    lens = PER - 1 - np.arange(PER)
    rows = np.repeat(np.arange(PER), lens)
    cols = np.concatenate([np.arange(r + 1, PER) for r in range(PER)])
    rlo = np.empty((WORKERS_PER_GRAPH,), dtype=np.int32)
    idx = np.empty((WORKERS_PER_GRAPH, OUT_PER_W), dtype=np.int32)
    for q in range(WORKERS_PER_GRAPH):
        sl = slice(q * OUT_PER_W, (q + 1) * OUT_PER_W)
        rlo[q] = rows[sl][0]
        assert rows[sl][-1] - rlo[q] < WIN_ROWS
        assert rlo[q] + WIN_ROWS <= PER
        idx[q] = (rows[sl] - rlo[q]) * PER + cols[sl]
    assert idx.min() >= 0 and idx.max() < WIN_ROWS * PER
    return rlo, idx


_RLO_NP, _IDX_NP = _build_triu_tables()


def _tc_body(t_ref, x_ref, wn_ref, bn_ref, w1_ref, b1_ref, w2_ref, b2_ref,
             sims_ref):
    a = jnp.dot(x_ref[...], wn_ref[...],
                preferred_element_type=jnp.float32) + bn_ref[...]
    half = TDIM // 2
    t_g = t_ref[0, 0, 0]
    freqs = jnp.exp(
        lax.broadcasted_iota(jnp.int32, (1, half), 1).astype(jnp.float32)
        * (-math.log(10000.0) / (half - 1)))
    ang = t_g * freqs
    pe = jnp.concatenate([jnp.sin(ang), jnp.cos(ang)], axis=1)
    h = jnp.dot(pe, w1_ref[...], preferred_element_type=jnp.float32) + b1_ref[...]
    h = h * 0.5 * (1.0 + lax.erf(h / np.sqrt(2.0).astype(np.float32)))
    temb = jnp.dot(h, w2_ref[...], preferred_element_type=jnp.float32) + b2_ref[...]
    c = jnp.sum(temb * temb)
    gram = lax.dot_general(a, a, (((1,), (1,)), ((), ())),
                           preferred_element_type=jnp.float32)
    sims_ref[...] = (gram + c)[None]


def _tc_sims(x, t, W_node, b_node, W1, b1, W2, b2):
    grid = (B,)
    return pl.pallas_call(
        _tc_body,
        grid=grid,
        in_specs=[
            pl.BlockSpec((1, 1, 1), lambda g: (g, 0, 0)),
            pl.BlockSpec((PER, FEAT), lambda g: (g, 0)),
            pl.BlockSpec((FEAT, TDIM), lambda g: (0, 0)),
            pl.BlockSpec((1, TDIM), lambda g: (0, 0)),
            pl.BlockSpec((TDIM, 4 * TDIM), lambda g: (0, 0)),
            pl.BlockSpec((1, 4 * TDIM), lambda g: (0, 0)),
            pl.BlockSpec((4 * TDIM, TDIM), lambda g: (0, 0)),
            pl.BlockSpec((1, TDIM), lambda g: (0, 0)),
        ],
        out_specs=pl.BlockSpec((1, PER, PER), lambda g: (g, 0, 0)),
        out_shape=jax.ShapeDtypeStruct((B, PER, PER), jnp.float32),
    )(t.reshape(B, 1, 1), x, W_node, b_node.reshape(1, TDIM),
      W1, b1.reshape(1, 4 * TDIM), W2, b2.reshape(1, TDIM))


def _sc_extract_body(sims_hbm, idx_hbm, out_hbm, rows_v, idx_v, out_v):
    wid = lax.axis_index("s") * 2 + lax.axis_index("c")
    g = wid // WORKERS_PER_GRAPH
    q = wid % WORKERS_PER_GRAPH
    rlo = jnp.where(
        q == 0, _RLO_NP[0],
        jnp.where(q == 1, _RLO_NP[1],
                  jnp.where(q == 2, _RLO_NP[2], _RLO_NP[3]))).astype(jnp.int32)
    src_off = g * (PER * PER) + rlo * PER
    pltpu.sync_copy(sims_hbm.at[pl.ds(src_off, WIN_ROWS * PER)], rows_v)
    pltpu.sync_copy(idx_hbm.at[q], idx_v)

    def body(i, _):
        iv = idx_v[pl.ds(i * 16, 16)]
        out_v[pl.ds(i * 16, 16)] = plsc.load_gather(rows_v, [iv])
        return 0

    lax.fori_loop(0, CHUNKS, body, 0)
    out_off = g * TRI + q * OUT_PER_W
    pltpu.sync_copy(out_v, out_hbm.at[pl.ds(out_off, OUT_PER_W)])


@functools.lru_cache(maxsize=None)
def _get_sc_extract():
    mesh = plsc.VectorSubcoreMesh(
        core_axis_name="c", subcore_axis_name="s",
        num_cores=2, num_subcores=16)
    return pl.kernel(
        _sc_extract_body,
        out_type=jax.ShapeDtypeStruct((B * TRI,), jnp.float32),
        mesh=mesh,
        compiler_params=pltpu.CompilerParams(needs_layout_passes=False),
        scratch_types=[
            pltpu.VMEM((WIN_ROWS * PER,), jnp.float32),
            pltpu.VMEM((OUT_PER_W,), jnp.int32),
            pltpu.VMEM((OUT_PER_W,), jnp.float32),
        ],
    )


def kernel(x, t, graph_sizes, W_node, b_node, W1, b1, W2, b2):
    sims = _tc_sims(x, t, W_node, b_node, W1, b1, W2, b2)
    idx = jnp.asarray(_IDX_NP)
    return _get_sc_extract()(sims.reshape(-1), idx)


# speedup vs baseline: 2.4207x; 1.1283x over previous
import functools
import math

import numpy as np
import jax
import jax.numpy as jnp
from jax import lax
from jax.experimental import pallas as pl
from jax.experimental.pallas import tpu as pltpu
from jax.experimental.pallas import tpu_sc as plsc

B = 8
PER = 256
N = B * PER
FEAT = 512
TDIM = 256
TRI = PER * (PER - 1) // 2
WORKERS_PER_GRAPH = 4
NUM_WORKERS = 32
OUT_PER_W = TRI // WORKERS_PER_GRAPH
CHUNKS = OUT_PER_W // 16
WIN_ROWS = 136


def _build_triu_tables():
    lens = PER - 1 - np.arange(PER)
    rows = np.repeat(np.arange(PER), lens)
    cols = np.concatenate([np.arange(r + 1, PER) for r in range(PER)])
    rlo = np.empty((WORKERS_PER_GRAPH,), dtype=np.int32)
    idx = np.empty((WORKERS_PER_GRAPH, OUT_PER_W), dtype=np.int32)
    for q in range(WORKERS_PER_GRAPH):
        sl = slice(q * OUT_PER_W, (q + 1) * OUT_PER_W)
        rlo[q] = (rows[sl][0] // 8) * 8
        assert rows[sl][-1] - rlo[q] < WIN_ROWS
        assert rlo[q] + WIN_ROWS <= PER
        idx[q] = (rows[sl] - rlo[q]) * PER + cols[sl]
    assert idx.min() >= 0 and idx.max() < WIN_ROWS * PER
    return rlo, idx


_RLO_NP, _IDX_NP = _build_triu_tables()


def _tc_body(t_ref, x_ref, wn_ref, bn_ref, w1_ref, b1_ref, w2_ref, b2_ref,
             sims_ref):
    a = jnp.dot(x_ref[...], wn_ref[...],
                preferred_element_type=jnp.float32) + bn_ref[...]
    half = TDIM // 2
    t_g = t_ref[0, 0, 0]
    freqs = jnp.exp(
        lax.broadcasted_iota(jnp.int32, (1, half), 1).astype(jnp.float32)
        * (-math.log(10000.0) / (half - 1)))
    ang = t_g * freqs
    pe = jnp.concatenate([jnp.sin(ang), jnp.cos(ang)], axis=1)
    h = jnp.dot(pe, w1_ref[...], preferred_element_type=jnp.float32) + b1_ref[...]
    h = h * 0.5 * (1.0 + lax.erf(h / np.sqrt(2.0).astype(np.float32)))
    temb = jnp.dot(h, w2_ref[...], preferred_element_type=jnp.float32) + b2_ref[...]
    c = jnp.sum(temb * temb)
    gram = lax.dot_general(a, a, (((1,), (1,)), ((), ())),
                           preferred_element_type=jnp.float32)
    sims_ref[...] = (gram + c)[None]


def _tc_sims(x, t, W_node, b_node, W1, b1, W2, b2):
    grid = (B,)
    return pl.pallas_call(
        _tc_body,
        grid=grid,
        in_specs=[
            pl.BlockSpec((1, 1, 1), lambda g: (g, 0, 0)),
            pl.BlockSpec((PER, FEAT), lambda g: (g, 0)),
            pl.BlockSpec((FEAT, TDIM), lambda g: (0, 0)),
            pl.BlockSpec((1, TDIM), lambda g: (0, 0)),
            pl.BlockSpec((TDIM, 4 * TDIM), lambda g: (0, 0)),
            pl.BlockSpec((1, 4 * TDIM), lambda g: (0, 0)),
            pl.BlockSpec((4 * TDIM, TDIM), lambda g: (0, 0)),
            pl.BlockSpec((1, TDIM), lambda g: (0, 0)),
        ],
        out_specs=pl.BlockSpec((1, PER, PER), lambda g: (g, 0, 0)),
        out_shape=jax.ShapeDtypeStruct((B, PER, PER), jnp.float32),
    )(t.reshape(B, 1, 1), x, W_node, b_node.reshape(1, TDIM),
      W1, b1.reshape(1, 4 * TDIM), W2, b2.reshape(1, TDIM))


def _sc_extract_body(sims_hbm, idx_hbm, out_hbm, rows_v, idx_v, out_v):
    wid = lax.axis_index("s") * 2 + lax.axis_index("c")
    g = wid // WORKERS_PER_GRAPH
    q = wid % WORKERS_PER_GRAPH
    rlo = jnp.where(
        q == 0, _RLO_NP[0],
        jnp.where(q == 1, _RLO_NP[1],
                  jnp.where(q == 2, _RLO_NP[2], _RLO_NP[3]))).astype(jnp.int32)
    rlo = pl.multiple_of(rlo, 8)
    pltpu.sync_copy(sims_hbm.at[g, pl.ds(rlo, WIN_ROWS), :], rows_v)
    pltpu.sync_copy(idx_hbm.at[q], idx_v)

    def body(i, _):
        iv = idx_v[pl.ds(i * 16, 16)]
        r = lax.shift_right_logical(iv, 8)
        c = lax.bitwise_and(iv, 255)
        out_v[pl.ds(i * 16, 16)] = plsc.load_gather(rows_v, [r, c])
        return 0

    lax.fori_loop(0, CHUNKS, body, 0)
    out_off = g * TRI + q * OUT_PER_W
    pltpu.sync_copy(out_v, out_hbm.at[pl.ds(out_off, OUT_PER_W)])


@functools.lru_cache(maxsize=None)
def _get_sc_extract():
    mesh = plsc.VectorSubcoreMesh(
        core_axis_name="c", subcore_axis_name="s",
        num_cores=2, num_subcores=16)
    return pl.kernel(
        _sc_extract_body,
        out_type=jax.ShapeDtypeStruct((B * TRI,), jnp.float32),
        mesh=mesh,
        compiler_params=pltpu.CompilerParams(needs_layout_passes=False),
        scratch_types=[
            pltpu.VMEM((WIN_ROWS, PER), jnp.float32),
            pltpu.VMEM((OUT_PER_W,), jnp.int32),
            pltpu.VMEM((OUT_PER_W,), jnp.float32),
        ],
    )


def kernel(x, t, graph_sizes, W_node, b_node, W1, b1, W2, b2):
    sims = _tc_sims(x, t, W_node, b_node, W1, b1, W2, b2)
    idx = jnp.asarray(_IDX_NP)
    return _get_sc_extract()(sims, idx)


# speedup vs baseline: 2.7186x; 1.1231x over previous
import functools
import math

import numpy as np
import jax
import jax.numpy as jnp
from jax import lax
from jax.experimental import pallas as pl
from jax.experimental.pallas import tpu as pltpu
from jax.experimental.pallas import tpu_sc as plsc

B = 8
PER = 256
N = B * PER
FEAT = 512
TDIM = 256
TRI = PER * (PER - 1) // 2
WORKERS_PER_GRAPH = 4
NUM_WORKERS = 32
OUT_PER_W = TRI // WORKERS_PER_GRAPH
CHUNKS = OUT_PER_W // 16
WIN_ROWS = 136


def _build_triu_tables():
    lens = PER - 1 - np.arange(PER)
    rows = np.repeat(np.arange(PER), lens)
    cols = np.concatenate([np.arange(r + 1, PER) for r in range(PER)])
    rlo = np.empty((WORKERS_PER_GRAPH,), dtype=np.int32)
    idx = np.empty((WORKERS_PER_GRAPH, OUT_PER_W), dtype=np.int32)
    for q in range(WORKERS_PER_GRAPH):
        sl = slice(q * OUT_PER_W, (q + 1) * OUT_PER_W)
        rlo[q] = (rows[sl][0] // 8) * 8
        assert rows[sl][-1] - rlo[q] < WIN_ROWS
        assert rlo[q] + WIN_ROWS <= PER
        idx[q] = (rows[sl] - rlo[q]) * PER + cols[sl]
    assert idx.min() >= 0 and idx.max() < WIN_ROWS * PER
    return rlo, idx


_RLO_NP, _IDX_NP = _build_triu_tables()


def _tc_body(t_ref, x_ref, wn_ref, bn_ref, w1_ref, b1_ref, w2_ref, b2_ref,
             sims_ref):
    a = jnp.dot(x_ref[...], wn_ref[...],
                preferred_element_type=jnp.float32) + bn_ref[...]
    half = TDIM // 2
    freqs = jnp.exp(
        lax.broadcasted_iota(jnp.int32, (1, half), 1).astype(jnp.float32)
        * (-math.log(10000.0) / (half - 1)))
    ang = t_ref[...] * freqs
    pe = jnp.concatenate([jnp.sin(ang), jnp.cos(ang)], axis=1)
    h = jnp.dot(pe, w1_ref[...], preferred_element_type=jnp.float32) + b1_ref[...]
    h = h * 0.5 * (1.0 + lax.erf(h / np.sqrt(2.0).astype(np.float32)))
    temb = jnp.dot(h, w2_ref[...], preferred_element_type=jnp.float32) + b2_ref[...]
    c = jnp.sum(temb * temb, axis=1, keepdims=True)
    for g in range(B):
        ag = a[g * PER:(g + 1) * PER, :]
        gram = lax.dot_general(ag, ag, (((1,), (1,)), ((), ())),
                               preferred_element_type=jnp.float32)
        sims_ref[g] = gram + c[g:g + 1, 0:1]


def _tc_sims(x, t, W_node, b_node, W1, b1, W2, b2):
    return pl.pallas_call(
        _tc_body,
        out_shape=jax.ShapeDtypeStruct((B, PER, PER), jnp.float32),
    )(t.reshape(B, 1), x, W_node, b_node.reshape(1, TDIM),
      W1, b1.reshape(1, 4 * TDIM), W2, b2.reshape(1, TDIM))


def _sc_extract_body(sims_hbm, idx_hbm, out_hbm, rows_v, idx_v, out_v):
    wid = lax.axis_index("s") * 2 + lax.axis_index("c")
    g = wid // WORKERS_PER_GRAPH
    q = wid % WORKERS_PER_GRAPH
    rlo = jnp.where(
        q == 0, _RLO_NP[0],
        jnp.where(q == 1, _RLO_NP[1],
                  jnp.where(q == 2, _RLO_NP[2], _RLO_NP[3]))).astype(jnp.int32)
    rlo = pl.multiple_of(rlo, 8)
    pltpu.sync_copy(sims_hbm.at[g, pl.ds(rlo, WIN_ROWS), :], rows_v)
    pltpu.sync_copy(idx_hbm.at[q], idx_v)

    def body(i, _):
        iv = idx_v[pl.ds(i * 16, 16)]
        r = lax.shift_right_logical(iv, 8)
        c = lax.bitwise_and(iv, 255)
        out_v[pl.ds(i * 16, 16)] = plsc.load_gather(rows_v, [r, c])
        return 0

    lax.fori_loop(0, CHUNKS, body, 0)
    out_off = g * TRI + q * OUT_PER_W
    pltpu.sync_copy(out_v, out_hbm.at[pl.ds(out_off, OUT_PER_W)])


@functools.lru_cache(maxsize=None)
def _get_sc_extract():
    mesh = plsc.VectorSubcoreMesh(
        core_axis_name="c", subcore_axis_name="s",
        num_cores=2, num_subcores=16)
    return pl.kernel(
        _sc_extract_body,
        out_type=jax.ShapeDtypeStruct((B * TRI,), jnp.float32),
        mesh=mesh,
        compiler_params=pltpu.CompilerParams(needs_layout_passes=False),
        scratch_types=[
            pltpu.VMEM((WIN_ROWS, PER), jnp.float32),
            pltpu.VMEM((OUT_PER_W,), jnp.int32),
            pltpu.VMEM((OUT_PER_W,), jnp.float32),
        ],
    )


def kernel(x, t, graph_sizes, W_node, b_node, W1, b1, W2, b2):
    sims = _tc_sims(x, t, W_node, b_node, W1, b1, W2, b2)
    idx = jnp.asarray(_IDX_NP)
    return _get_sc_extract()(sims, idx)


# speedup vs baseline: 2.9760x; 1.0947x over previous
import functools
import math

import numpy as np
import jax
import jax.numpy as jnp
from jax import lax
from jax.experimental import pallas as pl
from jax.experimental.pallas import tpu as pltpu
from jax.experimental.pallas import tpu_sc as plsc

B = 8
PER = 256
N = B * PER
FEAT = 512
TDIM = 256
TRI = PER * (PER - 1) // 2
WORKERS_PER_GRAPH = 4
NUM_WORKERS = 32
OUT_PER_W = TRI // WORKERS_PER_GRAPH
OUT_PAD = 8192
WIN_ROWS = 136


def _build_triu_tables():
    lens = PER - 1 - np.arange(PER)
    rows = np.repeat(np.arange(PER), lens)
    cols = np.concatenate([np.arange(r + 1, PER) for r in range(PER)])
    rlo = np.empty((WORKERS_PER_GRAPH,), dtype=np.int32)
    nrows = np.empty((WORKERS_PER_GRAPH,), dtype=np.int32)
    idx = np.zeros((WORKERS_PER_GRAPH, OUT_PAD), dtype=np.int32)
    for q in range(WORKERS_PER_GRAPH):
        sl = slice(q * OUT_PER_W, (q + 1) * OUT_PER_W)
        rlo[q] = (rows[sl][0] // 8) * 8
        nrows[q] = -((rlo[q] - (rows[sl][-1] + 1)) // 8) * 8
        assert rows[sl][-1] - rlo[q] < nrows[q] <= WIN_ROWS
        assert rlo[q] + nrows[q] <= PER
        idx[q, :OUT_PER_W] = (rows[sl] - rlo[q]) * PER + cols[sl]
    assert idx.min() >= 0 and idx.max() < WIN_ROWS * PER
    return rlo, nrows, idx


_RLO_NP, _NROWS_NP, _IDX_NP = _build_triu_tables()


def _tc_body(t_ref, x_ref, wn_ref, bn_ref, w1_ref, b1_ref, w2_ref, b2_ref,
             sims_ref):
    a = jnp.dot(x_ref[...], wn_ref[...],
                preferred_element_type=jnp.float32) + bn_ref[...]
    half = TDIM // 2
    freqs = jnp.exp(
        lax.broadcasted_iota(jnp.int32, (1, half), 1).astype(jnp.float32)
        * (-math.log(10000.0) / (half - 1)))
    ang = t_ref[...] * freqs
    pe = jnp.concatenate([jnp.sin(ang), jnp.cos(ang)], axis=1)
    h = jnp.dot(pe, w1_ref[...], preferred_element_type=jnp.float32) + b1_ref[...]
    h = h * 0.5 * (1.0 + lax.erf(h / np.sqrt(2.0).astype(np.float32)))
    temb = jnp.dot(h, w2_ref[...], preferred_element_type=jnp.float32) + b2_ref[...]
    c = jnp.sum(temb * temb, axis=1, keepdims=True)
    for g in range(B):
        ag = a[g * PER:(g + 1) * PER, :]
        gram = lax.dot_general(ag, ag, (((1,), (1,)), ((), ())),
                               preferred_element_type=jnp.float32)
        sims_ref[g] = gram + c[g:g + 1, 0:1]


def _tc_sims(x, t, W_node, b_node, W1, b1, W2, b2):
    return pl.pallas_call(
        _tc_body,
        out_shape=jax.ShapeDtypeStruct((B, PER, PER), jnp.float32),
    )(t.reshape(B, 1), x, W_node, b_node.reshape(1, TDIM),
      W1, b1.reshape(1, 4 * TDIM), W2, b2.reshape(1, TDIM))


def _sc_extract_body(sims_hbm, idx_hbm, out_hbm, rows_v, idx_v, out_v,
                     sem_rows, sem_idx):
    wid = lax.axis_index("s") * 2 + lax.axis_index("c")
    g = wid // WORKERS_PER_GRAPH
    q = wid % WORKERS_PER_GRAPH
    idx_cp = pltpu.async_copy(idx_hbm.at[q], idx_v, sem_idx)
    for qs in range(WORKERS_PER_GRAPH):
        @pl.when(q == qs)
        def _(qs=qs):
            nr = int(_NROWS_NP[qs])
            pltpu.async_copy(
                sims_hbm.at[g, pl.ds(int(_RLO_NP[qs]), nr), :],
                rows_v.at[pl.ds(0, nr), :], sem_rows).wait()
    idx_cp.wait()

    @plsc.parallel_loop(0, OUT_PAD, 16, unroll=8)
    def _(i):
        iv = idx_v[pl.ds(i, 16)]
        r = lax.shift_right_logical(iv, 8)
        c = lax.bitwise_and(iv, 255)
        out_v[pl.ds(i, 16)] = plsc.load_gather(rows_v, [r, c])

    out_off = g * TRI + q * OUT_PER_W
    pltpu.sync_copy(out_v.at[pl.ds(0, OUT_PER_W)],
                    out_hbm.at[pl.ds(out_off, OUT_PER_W)])


@functools.lru_cache(maxsize=None)
def _get_sc_extract():
    mesh = plsc.VectorSubcoreMesh(
        core_axis_name="c", subcore_axis_name="s",
        num_cores=2, num_subcores=16)
    return pl.kernel(
        _sc_extract_body,
        out_type=jax.ShapeDtypeStruct((B * TRI,), jnp.float32),
        mesh=mesh,
        compiler_params=pltpu.CompilerParams(needs_layout_passes=False),
        scratch_types=[
            pltpu.VMEM((WIN_ROWS, PER), jnp.float32),
            pltpu.VMEM((OUT_PAD,), jnp.int32),
            pltpu.VMEM((OUT_PAD,), jnp.float32),
            pltpu.SemaphoreType.DMA,
            pltpu.SemaphoreType.DMA,
        ],
    )


def kernel(x, t, graph_sizes, W_node, b_node, W1, b1, W2, b2):
    sims = _tc_sims(x, t, W_node, b_node, W1, b1, W2, b2)
    idx = jnp.asarray(_IDX_NP)
    return _get_sc_extract()(sims, idx)
